# Initial kernel scaffold; baseline (speedup 1.0000x reference)
import functools

import jax
import jax.numpy as jnp
from jax import lax
from jax.experimental import pallas as pl
from jax.experimental.pallas import tpu as pltpu
from jax.experimental.pallas import tpu_sc as plsc

B, F, T = 4, 256, 196
K = 512
TPAD = 256
BT = B * TPAD
NC, NS = 2, 16
NW = NC * NS
ROWS_PER_W = BT // NW


def _scores_body(x_ref, emb_ref, idx_ref, loss_ref):
    emb = emb_ref[...]
    xf = x_ref[...]
    s = jnp.dot(emb, xf, preferred_element_type=jnp.float32)
    csq = jnp.sum(emb * emb, axis=1, keepdims=True)
    m = csq - 2.0 * s
    mn = jnp.min(m, axis=0, keepdims=True)
    kio = lax.broadcasted_iota(jnp.int32, m.shape, 0)
    idx = jnp.min(jnp.where(m == mn, kio, K), axis=0, keepdims=True)
    xsq = jnp.sum(xf * xf, axis=0, keepdims=True)
    tio = lax.broadcasted_iota(jnp.int32, (1, BT), 1)
    valid = (tio % TPAD) < T
    dmin = (xsq + mn) * (1.0 / F)
    loss_ref[0, 0] = jnp.sum(jnp.where(valid, dmin, 0.0)) * (1.0 / (B * T))
    idx_ref[...] = idx


_scores = pl.pallas_call(
    _scores_body,
    out_shape=(
        jax.ShapeDtypeStruct((1, BT), jnp.int32),
        jax.ShapeDtypeStruct((1, 1), jnp.float32),
    ),
    in_specs=[
        pl.BlockSpec(memory_space=pltpu.VMEM),
        pl.BlockSpec(memory_space=pltpu.VMEM),
    ],
    out_specs=(
        pl.BlockSpec(memory_space=pltpu.VMEM),
        pl.BlockSpec(memory_space=pltpu.SMEM),
    ),
)


_mesh = plsc.VectorSubcoreMesh(
    core_axis_name="c", subcore_axis_name="s", num_cores=NC)


@functools.partial(
    pl.kernel,
    mesh=_mesh,
    out_type=jax.ShapeDtypeStruct((BT, F), jnp.float32),
    scratch_types=[
        pltpu.VMEM((ROWS_PER_W,), jnp.int32),
        pltpu.VMEM((ROWS_PER_W, F), jnp.float32),
        pltpu.SemaphoreType.DMA,
    ],
)
def _gather(table_hbm, idx_hbm, out_hbm, idx_v, rows_v, sem):
    wid = lax.axis_index("s") * NC + lax.axis_index("c")
    base = wid * ROWS_PER_W
    pltpu.sync_copy(idx_hbm.at[pl.ds(base, ROWS_PER_W)], idx_v)
    pltpu.async_copy(table_hbm.at[idx_v], rows_v, sem).wait()
    pltpu.sync_copy(rows_v, out_hbm.at[pl.ds(base, ROWS_PER_W)])


def kernel(x, emb_weight):
    xp = jnp.pad(x, ((0, 0), (0, 0), (0, TPAD - T)))
    x2d = jnp.transpose(xp, (1, 0, 2)).reshape(F, BT)
    idx, loss = _scores(x2d, emb_weight)
    rows = _gather(emb_weight, idx.reshape(BT))
    out = rows.reshape(B, TPAD, F)[:, :T, :].transpose(0, 2, 1)
    l = loss[0, 0]
    return (out, l, l)


# speedup vs baseline: 1.8712x; 1.8712x over previous
import functools

import jax
import jax.numpy as jnp
from jax import lax
from jax.experimental import pallas as pl
from jax.experimental.pallas import tpu as pltpu
from jax.experimental.pallas import tpu_sc as plsc

B, F, T = 4, 256, 196
K = 512
TPAD = 256
BT = B * TPAD
NC, NS = 2, 16
NW = NC * NS
ROWS_PER_W = BT // NW


def _scores_body(x_ref, emb_ref, idx_ref, loss_ref):
    emb = emb_ref[...]
    xf = x_ref[...]
    s = jnp.dot(emb, xf, preferred_element_type=jnp.float32,
                precision=lax.Precision.HIGHEST)
    csq = jnp.sum(emb * emb, axis=1, keepdims=True)
    m = csq - 2.0 * s
    mn = jnp.min(m, axis=0, keepdims=True)
    kio = lax.broadcasted_iota(jnp.int32, m.shape, 0)
    idx = jnp.min(jnp.where(m == mn, kio, K), axis=0, keepdims=True)
    xsq = jnp.sum(xf * xf, axis=0, keepdims=True)
    tio = lax.broadcasted_iota(jnp.int32, (1, BT), 1)
    valid = (tio % TPAD) < T
    dmin = (xsq + mn) * (1.0 / F)
    loss_ref[0, 0] = jnp.sum(jnp.where(valid, dmin, 0.0)) * (1.0 / (B * T))
    idx_ref[...] = idx


_scores = pl.pallas_call(
    _scores_body,
    out_shape=(
        jax.ShapeDtypeStruct((1, BT), jnp.int32),
        jax.ShapeDtypeStruct((1, 1), jnp.float32),
    ),
    in_specs=[
        pl.BlockSpec(memory_space=pltpu.VMEM),
        pl.BlockSpec(memory_space=pltpu.VMEM),
    ],
    out_specs=(
        pl.BlockSpec(memory_space=pltpu.VMEM),
        pl.BlockSpec(memory_space=pltpu.SMEM),
    ),
)


@functools.cache
def _make_gather():
    mesh = plsc.VectorSubcoreMesh(
        core_axis_name="c", subcore_axis_name="s", num_cores=NC)

    @functools.partial(
        pl.kernel,
        mesh=mesh,
        out_type=jax.ShapeDtypeStruct((BT, F), jnp.float32),
        scratch_types=[
            pltpu.VMEM((ROWS_PER_W,), jnp.int32),
            pltpu.VMEM((ROWS_PER_W, F), jnp.float32),
            pltpu.SemaphoreType.DMA,
        ],
    )
    def _gather(table_hbm, idx_hbm, out_hbm, idx_v, rows_v, sem):
        wid = lax.axis_index("s") * NC + lax.axis_index("c")
        base = wid * ROWS_PER_W
        pltpu.sync_copy(idx_hbm.at[pl.ds(base, ROWS_PER_W)], idx_v)
        pltpu.async_copy(table_hbm.at[idx_v], rows_v, sem).wait()
        pltpu.sync_copy(rows_v, out_hbm.at[pl.ds(base, ROWS_PER_W)])

    return _gather


def kernel(x, emb_weight):
    xp = jnp.pad(x, ((0, 0), (0, 0), (0, TPAD - T)))
    x2d = jnp.transpose(xp, (1, 0, 2)).reshape(F, BT)
    idx, loss = _scores(x2d, emb_weight)
    rows = _make_gather()(emb_weight, idx.reshape(BT))
    out = rows.reshape(B, TPAD, F)[:, :T, :].transpose(0, 2, 1)
    l = loss[0, 0]
    return (out, l, l)
